# baseline (speedup 1.0000x reference)
import functools

import jax
import jax.numpy as jnp
from jax import lax
from jax.experimental import pallas as pl
from jax.experimental.pallas import tpu as pltpu
from jax.experimental.pallas import tpu_sc as plsc

_NA = 3
_EMB = 64
_NR = 100
_CP = 128
_BT = 256
_BP = 4096


def _sc_gather_rows(table, idx, n_pad):
  info = plsc.get_sparse_core_info()
  nw = info.num_cores * info.num_subcores
  per = n_pad // nw
  mesh = plsc.VectorSubcoreMesh(core_axis_name="c", subcore_axis_name="s")

  @functools.partial(
      pl.kernel,
      mesh=mesh,
      out_type=jax.ShapeDtypeStruct((n_pad, _CP), jnp.float32),
      scratch_types=[
          pltpu.VMEM((per,), jnp.int32),
          pltpu.VMEM((per, _CP), jnp.float32),
          pltpu.SemaphoreType.DMA,
      ],
  )
  def gather(table_hbm, idx_hbm, out_hbm, idx_v, rows_v, sem):
    wid = lax.axis_index("s") * info.num_cores + lax.axis_index("c")
    base = wid * per
    pltpu.sync_copy(idx_hbm.at[pl.ds(base, per)], idx_v)
    pltpu.async_copy(table_hbm.at[idx_v], rows_v, sem).wait()
    pltpu.sync_copy(rows_v, out_hbm.at[pl.ds(base, per)])

  return gather(table, idx)


def _rgcn_tc_body(anch_ref, m_ref, tj_ref,
                  ct0_ref, ct1_ref, ct2_ref,
                  bf0_ref, bf1_ref, bf2_ref,
                  r0_ref, r1_ref, r2_ref,
                  b0_ref, b1_ref, b2_ref, out_ref):
  a = [jnp.transpose(anch_ref[j]) for j in range(_NA)]
  h = jnp.transpose(m_ref[...])[:_EMB]
  iota_r = lax.broadcasted_iota(jnp.int32, (_NR, _BT), 0)
  oh = [(tj_ref[j][None, :] == iota_r).astype(jnp.float32)
        for j in range(_NA)]
  ct_refs = (ct0_ref, ct1_ref, ct2_ref)
  bf_refs = (bf0_ref, bf1_ref, bf2_ref)
  r_refs = (r0_ref, r1_ref, r2_ref)
  b_refs = (b0_ref, b1_ref, b2_ref)
  tl = (((0,), (0,)), ((), ()))
  for l in range(3):
    ct = ct_refs[l][...]
    v = None
    for j in range(_NA):
      cj = lax.dot_general(ct, oh[j], tl,
                           preferred_element_type=jnp.float32)
      cjb = cj.astype(jnp.bfloat16)
      ajb = a[j].astype(jnp.bfloat16)
      t = cjb[:, None, :] * ajb[None, :, :]
      v = t if v is None else v + t
    agg = jnp.dot(bf_refs[l][...], v.reshape(_NR * _EMB, _BT),
                  preferred_element_type=jnp.float32)
    rl = r_refs[l][...]
    bias = b_refs[l][...][:, None]
    h = agg + lax.dot_general(rl, h, tl,
                              preferred_element_type=jnp.float32) + bias
    if l < 2:
      h = jnp.maximum(h, 0.0)
      a = [jnp.maximum(lax.dot_general(rl, a[j], tl,
                                       preferred_element_type=jnp.float32)
                       + bias, 0.0)
           for j in range(_NA)]
  out_ref[...] = h


def kernel(anchor_embeddings, var_ids, edge_index, edge_type, mode_emb,
           comp0, basis0, root0, bias0,
           comp1, basis1, root1, bias1,
           comp2, basis2, root2, bias2):
  del edge_index
  b = anchor_embeddings.shape[1]

  table = jnp.pad(mode_emb, ((0, 0), (0, _CP - _EMB)))
  vid = jnp.pad(var_ids[:, 0].astype(jnp.int32), (0, _BP - b))
  m_rows = _sc_gather_rows(table, vid, _BP)

  tj = jnp.pad(edge_type.astype(jnp.int32).reshape(b, _NA).T,
               ((0, 0), (0, _BP - b)))

  bfs = [x.transpose(2, 0, 1).reshape(_EMB, _NR * _EMB).astype(jnp.bfloat16)
         for x in (basis0, basis1, basis2)]
  wspec = lambda shape: pl.BlockSpec(shape, lambda g: tuple(0 for _ in shape))
  out = pl.pallas_call(
      _rgcn_tc_body,
      grid=(_BP // _BT,),
      in_specs=[
          pl.BlockSpec((_NA, _BT, _EMB), lambda g: (0, g, 0)),
          pl.BlockSpec((_BT, _CP), lambda g: (g, 0)),
          pl.BlockSpec((_NA, _BT), lambda g: (0, g)),
          wspec((_NR, _NR)),
          wspec((_NR, _NR)),
          wspec((_NR, _NR)),
          wspec((_EMB, _NR * _EMB)),
          wspec((_EMB, _NR * _EMB)),
          wspec((_EMB, _NR * _EMB)),
          wspec((_EMB, _EMB)),
          wspec((_EMB, _EMB)),
          wspec((_EMB, _EMB)),
          wspec((_EMB,)),
          wspec((_EMB,)),
          wspec((_EMB,)),
      ],
      out_specs=pl.BlockSpec((_EMB, _BT), lambda g: (0, g)),
      out_shape=jax.ShapeDtypeStruct((_EMB, _BP), jnp.float32),
  )(anchor_embeddings, m_rows, tj,
    comp0, comp1, comp2,
    bfs[0], bfs[1], bfs[2], root0, root1, root2,
    bias0, bias1, bias2)
  return out[:, :b].T


# speedup vs baseline: 1.0146x; 1.0146x over previous
import functools

import jax
import jax.numpy as jnp
from jax import lax
from jax.experimental import pallas as pl
from jax.experimental.pallas import tpu as pltpu
from jax.experimental.pallas import tpu_sc as plsc

_NA = 3
_EMB = 64
_NR = 100
_CP = 128
_BT = 256
_BP = 4096


def _sc_gather_rows(table, idx, n_pad):
  info = plsc.get_sparse_core_info()
  nw = info.num_cores * info.num_subcores
  per = n_pad // nw
  mesh = plsc.VectorSubcoreMesh(core_axis_name="c", subcore_axis_name="s")

  @functools.partial(
      pl.kernel,
      mesh=mesh,
      out_type=jax.ShapeDtypeStruct((n_pad, _CP), jnp.float32),
      scratch_types=[
          pltpu.VMEM((per,), jnp.int32),
          pltpu.VMEM((per, _CP), jnp.float32),
          pltpu.SemaphoreType.DMA,
      ],
  )
  def gather(table_hbm, idx_hbm, out_hbm, idx_v, rows_v, sem):
    wid = lax.axis_index("s") * info.num_cores + lax.axis_index("c")
    base = wid * per
    pltpu.sync_copy(idx_hbm.at[pl.ds(base, per)], idx_v)
    pltpu.async_copy(table_hbm.at[idx_v], rows_v, sem).wait()
    pltpu.sync_copy(rows_v, out_hbm.at[pl.ds(base, per)])

  return gather(table, idx)


def _rgcn_tc_body(anch_ref, m_ref, tj_ref,
                  ct0_ref, ct1_ref, ct2_ref,
                  bf0_ref, bf1_ref, bf2_ref,
                  r0_ref, r1_ref, r2_ref,
                  b0_ref, b1_ref, b2_ref, out_ref):
  a = [jnp.transpose(anch_ref[j]) for j in range(_NA)]
  h = jnp.transpose(m_ref[...])[:_EMB]
  iota_r = lax.broadcasted_iota(jnp.int32, (_NR, _BT), 0)
  oh = [(tj_ref[j][None, :] == iota_r).astype(jnp.float32)
        for j in range(_NA)]
  ct_refs = (ct0_ref, ct1_ref, ct2_ref)
  bf_refs = (bf0_ref, bf1_ref, bf2_ref)
  r_refs = (r0_ref, r1_ref, r2_ref)
  b_refs = (b0_ref, b1_ref, b2_ref)
  for l in range(3):
    ct = ct_refs[l][...]
    v = None
    for j in range(_NA):
      cj = jnp.dot(ct, oh[j], preferred_element_type=jnp.float32)
      cjb = cj.astype(jnp.bfloat16)
      ajb = a[j].astype(jnp.bfloat16)
      t = cjb[:, None, :] * ajb[None, :, :]
      v = t if v is None else v + t
    agg = jnp.dot(bf_refs[l][...], v.reshape(_NR * _EMB, _BT),
                  preferred_element_type=jnp.float32)
    rl = r_refs[l][...]
    bias = b_refs[l][...]
    h = agg + jnp.dot(rl, h, preferred_element_type=jnp.float32) + bias
    if l < 2:
      h = jnp.maximum(h, 0.0)
      a = [jnp.maximum(jnp.dot(rl, a[j], preferred_element_type=jnp.float32)
                       + bias, 0.0)
           for j in range(_NA)]
  out_ref[...] = h


def kernel(anchor_embeddings, var_ids, edge_index, edge_type, mode_emb,
           comp0, basis0, root0, bias0,
           comp1, basis1, root1, bias1,
           comp2, basis2, root2, bias2):
  del edge_index
  b = anchor_embeddings.shape[1]

  table = jnp.pad(mode_emb, ((0, 0), (0, _CP - _EMB)))
  vid = jnp.pad(var_ids[:, 0].astype(jnp.int32), (0, _BP - b))
  m_rows = _sc_gather_rows(table, vid, _BP)

  tj = jnp.pad(edge_type.astype(jnp.int32).reshape(b, _NA).T,
               ((0, 0), (0, _BP - b)))

  cts = [x.T for x in (comp0, comp1, comp2)]
  bfs = [x.transpose(2, 0, 1).reshape(_EMB, _NR * _EMB).astype(jnp.bfloat16)
         for x in (basis0, basis1, basis2)]
  roots_t = [x.T for x in (root0, root1, root2)]
  biases = [x.reshape(_EMB, 1) for x in (bias0, bias1, bias2)]
  wspec = lambda shape: pl.BlockSpec(shape, lambda g: tuple(0 for _ in shape))
  out = pl.pallas_call(
      _rgcn_tc_body,
      grid=(_BP // _BT,),
      in_specs=[
          pl.BlockSpec((_NA, _BT, _EMB), lambda g: (0, g, 0)),
          pl.BlockSpec((_BT, _CP), lambda g: (g, 0)),
          pl.BlockSpec((_NA, _BT), lambda g: (0, g)),
          wspec((_NR, _NR)),
          wspec((_NR, _NR)),
          wspec((_NR, _NR)),
          wspec((_EMB, _NR * _EMB)),
          wspec((_EMB, _NR * _EMB)),
          wspec((_EMB, _NR * _EMB)),
          wspec((_EMB, _EMB)),
          wspec((_EMB, _EMB)),
          wspec((_EMB, _EMB)),
          wspec((_EMB, 1)),
          wspec((_EMB, 1)),
          wspec((_EMB, 1)),
      ],
      out_specs=pl.BlockSpec((_EMB, _BT), lambda g: (0, g)),
      out_shape=jax.ShapeDtypeStruct((_EMB, _BP), jnp.float32),
  )(anchor_embeddings, m_rows, tj,
    cts[0], cts[1], cts[2],
    bfs[0], bfs[1], bfs[2], roots_t[0], roots_t[1], roots_t[2],
    biases[0], biases[1], biases[2])
  return out[:, :b].T
